# baseline (speedup 1.0000x reference)
import functools

import jax
import jax.numpy as jnp
from jax import lax
from jax.experimental import pallas as pl
from jax.experimental.pallas import tpu as pltpu
from jax.experimental.pallas import tpu_sc as plsc

B, V, N, D, K = 2, 4, 196, 384, 8
NV = V - 1
M = V * N
NP = 256
D2 = 256


def _scores_body(xq_ref, xk_ref, out_ref):
    xq = xq_ref[0, 0]
    xk = xk_ref[0, 0]
    gram = jax.lax.dot_general(xq, xk, (((1,), (1,)), ((), ())),
                               preferred_element_type=jnp.float32)
    ones_row = jnp.ones((8, D), jnp.float32)
    ss = jax.lax.dot_general(ones_row, xk * xk, (((1,), (1,)), ((), ())),
                             preferred_element_type=jnp.float32)[:1]
    s = ss - 2.0 * gram
    pad = jnp.full((N, NP - N), jnp.inf, jnp.float32)
    out_ref[0, 0, 0] = jnp.concatenate([s, pad], axis=1)


def _scores(x):
    grid = (B, V, NV)
    return pl.pallas_call(
        _scores_body,
        grid=grid,
        in_specs=[
            pl.BlockSpec((1, 1, N, D), lambda b, v, j: (b, v, 0, 0)),
            pl.BlockSpec((1, 1, N, D),
                         lambda b, v, j: (b, j + (j >= v).astype(j.dtype), 0, 0)),
        ],
        out_specs=pl.BlockSpec((1, 1, 1, N, NP),
                               lambda b, v, j: (b, v, j, 0, 0)),
        out_shape=jax.ShapeDtypeStruct((B, V, NV, N, NP), jnp.float32),
    )(x, x)


def _proj_body(x_ref, we_ref, wo1_ref, bo_ref, u_ref, u16_ref, p_ref):
    xb = x_ref[0, 0]
    u = jax.lax.dot_general(xb, we_ref[...], (((1,), (0,)), ((), ())),
                            preferred_element_type=jnp.float32)
    u_ref[0, 0] = u
    u16_ref[0, 0] = jnp.concatenate(
        [u.astype(jnp.bfloat16), jnp.zeros((N, 2 * D2 - D), jnp.bfloat16)],
        axis=1)
    p_ref[0, 0] = xb + jax.lax.dot_general(
        xb, wo1_ref[...], (((1,), (0,)), ((), ())),
        preferred_element_type=jnp.float32) + bo_ref[...][None, :]


def _proj(x, W_edge, Wo1, b_offset):
    grid = (B, V)
    return pl.pallas_call(
        _proj_body,
        grid=grid,
        in_specs=[
            pl.BlockSpec((1, 1, N, D), lambda b, v: (b, v, 0, 0)),
            pl.BlockSpec((D, D), lambda b, v: (0, 0)),
            pl.BlockSpec((D, D), lambda b, v: (0, 0)),
            pl.BlockSpec((D,), lambda b, v: (0,)),
        ],
        out_specs=[
            pl.BlockSpec((1, 1, N, D), lambda b, v: (b, v, 0, 0)),
            pl.BlockSpec((1, 1, N, 2 * D2), lambda b, v: (b, v, 0, 0)),
            pl.BlockSpec((1, 1, N, D), lambda b, v: (b, v, 0, 0)),
        ],
        out_shape=[
            jax.ShapeDtypeStruct((B, V, N, D), jnp.float32),
            jax.ShapeDtypeStruct((B, V, N, 2 * D2), jnp.bfloat16),
            jax.ShapeDtypeStruct((B, V, N, D), jnp.float32),
        ],
    )(x, W_edge, Wo1, b_offset)


QT = 49
NCH = NP // 16
GQ = 4


def _sc_topk_gather_body(scores_hbm, u_hbm, ug_hbm, b0, b1, b2, idx_v,
                         rows_v, rows_tail, sem):
    nc = 2
    wid = lax.axis_index("s") * nc + lax.axis_index("c")
    q0 = wid * QT
    b = q0 // M
    v = (q0 % M) // N
    n0 = q0 % N
    bufs = (b0, b1, b2)
    offs = []
    for j in range(NV):
        row0 = ((b * V + v) * NV + j) * N + n0
        al = (row0 // 8) * 8
        offs.append(row0 - al)
        pltpu.sync_copy(scores_hbm.at[pl.ds(al, QT + 7)], bufs[j])

    lane = lax.iota(jnp.int32, 16)
    inf16 = jnp.full((16,), jnp.inf, jnp.float32)
    zero16 = jnp.zeros((16,), jnp.int32)
    sentinel = jnp.where(lane < K, 0, jnp.int32(2**30))

    def per_query(qq, _):
        def per_chunk(c, st):
            out = []
            for j in range(NV):
                bv, bi = st[2 * j], st[2 * j + 1]
                vals = bufs[j][qq + offs[j], pl.ds(c * 16, 16)]
                idxs = c * 16 + lane
                sv, si = plsc.sort_key_val(vals, idxs)
                rv = lax.rev(sv, (0,))
                ri = lax.rev(si, (0,))
                keep = bv <= rv
                mv = jnp.where(keep, bv, rv)
                mi = jnp.where(keep, bi, ri)
                nbv, nbi = plsc.sort_key_val(mv, mi)
                out.extend([nbv, nbi])
            return tuple(out)

        st = (inf16, zero16) * NV
        st = lax.fori_loop(0, NCH, per_chunk, st)
        for j in range(NV):
            v1 = j + jnp.where(j >= v, 1, 0)
            base = b * M + v1 * N
            keys = st[2 * j + 1] + sentinel
            gidx, _ = plsc.sort_key_val(keys, keys)
            plsc.store_compressed(idx_v.at[pl.ds(qq * (NV * K) + j * K, 16)],
                                  gidx + base, mask=lane < K)
        return 0

    lax.fori_loop(0, QT, per_query, 0)

    out0 = q0 * NV * K
    for ch in range(QT // GQ):
        cp = pltpu.async_copy(
            u_hbm.at[idx_v.at[pl.ds(ch * GQ * NV * K, GQ * NV * K)]],
            rows_v, sem)
        cp.wait()
        pltpu.sync_copy(rows_v, ug_hbm.at[pl.ds(out0 + ch * GQ * NV * K,
                                                GQ * NV * K)])
    tail0 = (QT // GQ) * GQ * NV * K
    cp = pltpu.async_copy(u_hbm.at[idx_v.at[pl.ds(tail0, NV * K)]],
                          rows_tail, sem)
    cp.wait()
    pltpu.sync_copy(rows_tail, ug_hbm.at[pl.ds(out0 + tail0, NV * K)])


def _topk_gather_sc(scores, U16):
    scores2 = scores.reshape(B * V * NV * N, NP)
    U2 = jax.lax.bitcast_convert_type(
        U16.reshape(B * M, D2, 2), jnp.int32)
    mesh = plsc.VectorSubcoreMesh(core_axis_name="c", subcore_axis_name="s")
    fn = pl.kernel(
        _sc_topk_gather_body,
        out_type=jax.ShapeDtypeStruct((B * M * NV * K, D2), jnp.int32),
        mesh=mesh,
        scratch_types=[
            pltpu.VMEM((QT + 7, NP), jnp.float32),
            pltpu.VMEM((QT + 7, NP), jnp.float32),
            pltpu.VMEM((QT + 7, NP), jnp.float32),
            pltpu.VMEM((QT * NV * K + 8,), jnp.int32),
            pltpu.VMEM((GQ * NV * K, D2), jnp.int32),
            pltpu.VMEM((NV * K, D2), jnp.int32),
            pltpu.SemaphoreType.DMA,
        ],
        compiler_params=pltpu.CompilerParams(needs_layout_passes=False),
    )
    return fn(scores2, U2)


QC = 49


def _fusion_body(ug_ref, u_ref, p_ref, wf_ref, wo2_ref, be_ref, out_ref):
    ug = ug_ref[0][:, :D].astype(jnp.float32)
    uq = u_ref[0]
    pre = (ug.reshape(QC, NV * K, D) - uq[:, None, :] + be_ref[...][None, None, :])
    edge = 0.5 * pre * (1.0 + lax.erf(pre * (2.0 ** -0.5)))
    logits = jax.lax.dot_general(
        edge.reshape(QC * NV * K, D), wf_ref[...], (((1,), (0,)), ((), ())),
        preferred_element_type=jnp.float32).reshape(QC, NV * K, D)
    mx = jnp.max(logits, axis=1, keepdims=True)
    e = jnp.exp(logits - mx)
    edge_sum = jnp.sum(edge * e, axis=1) / jnp.sum(e, axis=1)
    out_ref[0] = p_ref[0] + jax.lax.dot_general(
        edge_sum, wo2_ref[...], (((1,), (0,)), ((), ())),
        preferred_element_type=jnp.float32)


def _fusion(Ug, U, P, W_fusion, Wo2, b_edge):
    R = B * M
    grid = (R // QC,)
    return pl.pallas_call(
        _fusion_body,
        grid=grid,
        in_specs=[
            pl.BlockSpec((1, QC * NV * K, 2 * D2), lambda i: (i, 0, 0)),
            pl.BlockSpec((1, QC, D), lambda i: (i, 0, 0)),
            pl.BlockSpec((1, QC, D), lambda i: (i, 0, 0)),
            pl.BlockSpec((D, D), lambda i: (0, 0)),
            pl.BlockSpec((D, D), lambda i: (0, 0)),
            pl.BlockSpec((D,), lambda i: (0,)),
        ],
        out_specs=pl.BlockSpec((1, QC, D), lambda i: (i, 0, 0)),
        out_shape=jax.ShapeDtypeStruct((R // QC, QC, D), jnp.float32),
    )(Ug.reshape(R // QC, QC * NV * K, 2 * D2),
      U.reshape(R // QC, QC, D),
      P.reshape(R // QC, QC, D),
      W_fusion, Wo2, b_edge)


def kernel(x, W_edge, b_edge, W_fusion, W_offset, b_offset):
    scores = _scores(x)
    U, U16, P = _proj(x, W_edge, W_offset[:D], b_offset)
    Ug_i32 = _topk_gather_sc(scores, U16)
    Ug = jax.lax.bitcast_convert_type(Ug_i32, jnp.bfloat16).reshape(
        B * M * NV * K, 2 * D2)
    out = _fusion(Ug,
                  U.reshape(B * M, D),
                  P.reshape(B * M, D),
                  W_fusion, W_offset[D:], b_edge)
    return out.reshape(B, V, N, D)


# speedup vs baseline: 2.6475x; 2.6475x over previous
import functools

import jax
import jax.numpy as jnp
from jax import lax
from jax.experimental import pallas as pl
from jax.experimental.pallas import tpu as pltpu
from jax.experimental.pallas import tpu_sc as plsc

B, V, N, D, K = 2, 4, 196, 384, 8
NV = V - 1
M = V * N
NP = 256
D2 = 256


def _scores_proj_body(xq_ref, xk_ref, we_ref, wo1_ref, bo_ref,
                      out_ref, u_ref, p_ref):
    xq = xq_ref[0, 0]
    xk = xk_ref[0, 0]
    gram = jax.lax.dot_general(xq, xk, (((1,), (1,)), ((), ())),
                               preferred_element_type=jnp.float32)
    ones_row = jnp.ones((8, D), jnp.float32)
    ss = jax.lax.dot_general(ones_row, xk * xk, (((1,), (1,)), ((), ())),
                             preferred_element_type=jnp.float32)[:1]
    s = ss - 2.0 * gram
    pad = jnp.full((N, NP - N), jnp.inf, jnp.float32)
    out_ref[0, 0, 0] = jnp.concatenate([s, pad], axis=1)

    @pl.when(pl.program_id(2) == 0)
    def _():
        u_ref[0, 0] = jax.lax.dot_general(
            xq, we_ref[...], (((1,), (0,)), ((), ())),
            preferred_element_type=jnp.float32)
        p_ref[0, 0] = xq + jax.lax.dot_general(
            xq, wo1_ref[...], (((1,), (0,)), ((), ())),
            preferred_element_type=jnp.float32) + bo_ref[...][None, :]


def _scores_proj(x, W_edge, Wo1, b_offset):
    grid = (B, V, NV)
    return pl.pallas_call(
        _scores_proj_body,
        grid=grid,
        in_specs=[
            pl.BlockSpec((1, 1, N, D), lambda b, v, j: (b, v, 0, 0)),
            pl.BlockSpec((1, 1, N, D),
                         lambda b, v, j: (b, j + (j >= v).astype(j.dtype), 0, 0)),
            pl.BlockSpec((D, D), lambda b, v, j: (0, 0)),
            pl.BlockSpec((D, D), lambda b, v, j: (0, 0)),
            pl.BlockSpec((D,), lambda b, v, j: (0,)),
        ],
        out_specs=[
            pl.BlockSpec((1, 1, 1, N, NP), lambda b, v, j: (b, v, j, 0, 0)),
            pl.BlockSpec((1, 1, N, D), lambda b, v, j: (b, v, 0, 0)),
            pl.BlockSpec((1, 1, N, D), lambda b, v, j: (b, v, 0, 0)),
        ],
        out_shape=[
            jax.ShapeDtypeStruct((B, V, NV, N, NP), jnp.float32),
            jax.ShapeDtypeStruct((B, V, N, D), jnp.float32),
            jax.ShapeDtypeStruct((B, V, N, D), jnp.float32),
        ],
    )(x, x, W_edge, Wo1, b_offset)


QT = 49
NCH = NP // 16
GQ = 4


def _sc_topk_gather_body(scores_hbm, u_hbm, ug_hbm, b0, b1, b2, idx_v,
                         rows_v, rows_tail, sem):
    nc = 2
    wid = lax.axis_index("s") * nc + lax.axis_index("c")
    q0 = wid * QT
    b = q0 // M
    v = (q0 % M) // N
    n0 = q0 % N
    bufs = (b0, b1, b2)
    offs = []
    for j in range(NV):
        row0 = ((b * V + v) * NV + j) * N + n0
        al = (row0 // 8) * 8
        offs.append(row0 - al)
        pltpu.sync_copy(scores_hbm.at[pl.ds(al, QT + 7)], bufs[j])

    lane = lax.iota(jnp.int32, 16)
    inf16 = jnp.full((16,), jnp.inf, jnp.float32)
    zero16 = jnp.zeros((16,), jnp.int32)
    sentinel = jnp.where(lane < K, 0, jnp.int32(2**30))

    def per_query(qq, _):
        def per_chunk(c, st):
            out = []
            for j in range(NV):
                bv, bi = st[2 * j], st[2 * j + 1]
                vals = bufs[j][qq + offs[j], pl.ds(c * 16, 16)]
                idxs = c * 16 + lane
                sv, si = plsc.sort_key_val(vals, idxs)
                rv = lax.rev(sv, (0,))
                ri = lax.rev(si, (0,))
                keep = bv <= rv
                mv = jnp.where(keep, bv, rv)
                mi = jnp.where(keep, bi, ri)
                nbv, nbi = plsc.sort_key_val(mv, mi)
                out.extend([nbv, nbi])
            return tuple(out)

        st = (inf16, zero16) * NV
        for c in range(NCH):
            st = per_chunk(c, st)
        for j in range(NV):
            v1 = j + jnp.where(j >= v, 1, 0)
            base = b * M + v1 * N
            keys = st[2 * j + 1] + sentinel
            gidx, _ = plsc.sort_key_val(keys, keys)
            plsc.store_compressed(idx_v.at[pl.ds(qq * (NV * K) + j * K, 16)],
                                  gidx + base, mask=lane < K)
        return 0

    lax.fori_loop(0, QT, per_query, 0)

    out0 = q0 * NV * K
    for ch in range(QT // GQ):
        cp = pltpu.async_copy(
            u_hbm.at[idx_v.at[pl.ds(ch * GQ * NV * K, GQ * NV * K)]],
            rows_v, sem)
        cp.wait()
        pltpu.sync_copy(rows_v, ug_hbm.at[pl.ds(out0 + ch * GQ * NV * K,
                                                GQ * NV * K)])
    tail0 = (QT // GQ) * GQ * NV * K
    cp = pltpu.async_copy(u_hbm.at[idx_v.at[pl.ds(tail0, NV * K)]],
                          rows_tail, sem)
    cp.wait()
    pltpu.sync_copy(rows_tail, ug_hbm.at[pl.ds(out0 + tail0, NV * K)])


def _topk_gather_sc(scores, U):
    scores2 = scores.reshape(B * V * NV * N, NP)
    U2 = U.reshape(B * M, D)
    mesh = plsc.VectorSubcoreMesh(core_axis_name="c", subcore_axis_name="s")
    fn = pl.kernel(
        _sc_topk_gather_body,
        out_type=jax.ShapeDtypeStruct((B * M * NV * K, D), jnp.float32),
        mesh=mesh,
        scratch_types=[
            pltpu.VMEM((QT + 7, NP), jnp.float32),
            pltpu.VMEM((QT + 7, NP), jnp.float32),
            pltpu.VMEM((QT + 7, NP), jnp.float32),
            pltpu.VMEM((QT * NV * K + 8,), jnp.int32),
            pltpu.VMEM((GQ * NV * K, D), jnp.float32),
            pltpu.VMEM((NV * K, D), jnp.float32),
            pltpu.SemaphoreType.DMA,
        ],
        compiler_params=pltpu.CompilerParams(needs_layout_passes=False),
    )
    return fn(scores2, U2)


QC = 49


def _fusion_body(ug_ref, u_ref, p_ref, wf_ref, wo2_ref, be_ref, out_ref):
    ug = ug_ref[0]
    uq = u_ref[0]
    pre = (ug.reshape(QC, NV * K, D) - uq[:, None, :] + be_ref[...][None, None, :])
    edge = 0.5 * pre * (1.0 + lax.erf(pre * (2.0 ** -0.5)))
    logits = jax.lax.dot_general(
        edge.reshape(QC * NV * K, D), wf_ref[...], (((1,), (0,)), ((), ())),
        preferred_element_type=jnp.float32).reshape(QC, NV * K, D)
    mx = jnp.max(logits, axis=1, keepdims=True)
    e = jnp.exp(logits - mx)
    edge_sum = jnp.sum(edge * e, axis=1) / jnp.sum(e, axis=1)
    out_ref[0] = p_ref[0] + jax.lax.dot_general(
        edge_sum, wo2_ref[...], (((1,), (0,)), ((), ())),
        preferred_element_type=jnp.float32)


def _fusion(Ug, U, P, W_fusion, Wo2, b_edge):
    R = B * M
    grid = (R // QC,)
    return pl.pallas_call(
        _fusion_body,
        grid=grid,
        in_specs=[
            pl.BlockSpec((1, QC * NV * K, D), lambda i: (i, 0, 0)),
            pl.BlockSpec((1, QC, D), lambda i: (i, 0, 0)),
            pl.BlockSpec((1, QC, D), lambda i: (i, 0, 0)),
            pl.BlockSpec((D, D), lambda i: (0, 0)),
            pl.BlockSpec((D, D), lambda i: (0, 0)),
            pl.BlockSpec((D,), lambda i: (0,)),
        ],
        out_specs=pl.BlockSpec((1, QC, D), lambda i: (i, 0, 0)),
        out_shape=jax.ShapeDtypeStruct((R // QC, QC, D), jnp.float32),
    )(Ug.reshape(R // QC, QC * NV * K, D),
      U.reshape(R // QC, QC, D),
      P.reshape(R // QC, QC, D),
      W_fusion, Wo2, b_edge)


def kernel(x, W_edge, b_edge, W_fusion, W_offset, b_offset):
    scores, U, P = _scores_proj(x, W_edge, W_offset[:D], b_offset)
    Ug = _topk_gather_sc(scores, U)
    out = _fusion(Ug,
                  U.reshape(B * M, D),
                  P.reshape(B * M, D),
                  W_fusion, W_offset[D:], b_edge)
    return out.reshape(B, V, N, D)


# speedup vs baseline: 2.6615x; 1.0053x over previous
import functools

import jax
import jax.numpy as jnp
from jax import lax
from jax.experimental import pallas as pl
from jax.experimental.pallas import tpu as pltpu
from jax.experimental.pallas import tpu_sc as plsc

B, V, N, D, K = 2, 4, 196, 384, 8
NV = V - 1
M = V * N
NP = 256
D2 = 256


def _scores_proj_body(xq_ref, xk_ref, we_ref, wo1_ref, bo_ref,
                      out_ref, u_ref, p_ref):
    xq = xq_ref[0, 0]
    xk = xk_ref[0, 0]
    gram = jax.lax.dot_general(xq, xk, (((1,), (1,)), ((), ())),
                               preferred_element_type=jnp.float32)
    ones_row = jnp.ones((8, D), jnp.float32)
    ss = jax.lax.dot_general(ones_row, xk * xk, (((1,), (1,)), ((), ())),
                             preferred_element_type=jnp.float32)[:1]
    s = ss - 2.0 * gram
    pad = jnp.full((N, NP - N), jnp.inf, jnp.float32)
    out_ref[0, 0, 0] = jnp.concatenate([s, pad], axis=1)

    @pl.when(pl.program_id(2) == 0)
    def _():
        u_ref[0, 0] = jax.lax.dot_general(
            xq, we_ref[...], (((1,), (0,)), ((), ())),
            preferred_element_type=jnp.float32)
        p_ref[0, 0] = xq + jax.lax.dot_general(
            xq, wo1_ref[...], (((1,), (0,)), ((), ())),
            preferred_element_type=jnp.float32) + bo_ref[...][None, :]


def _scores_proj(x, W_edge, Wo1, b_offset):
    grid = (B, V, NV)
    return pl.pallas_call(
        _scores_proj_body,
        grid=grid,
        in_specs=[
            pl.BlockSpec((1, 1, N, D), lambda b, v, j: (b, v, 0, 0)),
            pl.BlockSpec((1, 1, N, D),
                         lambda b, v, j: (b, j + (j >= v).astype(j.dtype), 0, 0)),
            pl.BlockSpec((D, D), lambda b, v, j: (0, 0)),
            pl.BlockSpec((D, D), lambda b, v, j: (0, 0)),
            pl.BlockSpec((D,), lambda b, v, j: (0,)),
        ],
        out_specs=[
            pl.BlockSpec((1, 1, 1, N, NP), lambda b, v, j: (b, v, j, 0, 0)),
            pl.BlockSpec((1, 1, N, D), lambda b, v, j: (b, v, 0, 0)),
            pl.BlockSpec((1, 1, N, D), lambda b, v, j: (b, v, 0, 0)),
        ],
        out_shape=[
            jax.ShapeDtypeStruct((B, V, NV, N, NP), jnp.float32),
            jax.ShapeDtypeStruct((B, V, N, D), jnp.float32),
            jax.ShapeDtypeStruct((B, V, N, D), jnp.float32),
        ],
    )(x, x, W_edge, Wo1, b_offset)


QT = 49
NCH = NP // 16
GQ = 4


def _sc_topk_gather_body(scores_hbm, u_hbm, ug_hbm, b0, b1, b2, idx_v,
                         rows_v, rows_tail, sem):
    nc = 2
    wid = lax.axis_index("s") * nc + lax.axis_index("c")
    q0 = wid * QT
    b = q0 // M
    v = (q0 % M) // N
    n0 = q0 % N
    bufs = (b0, b1, b2)
    offs = []
    for j in range(NV):
        row0 = ((b * V + v) * NV + j) * N + n0
        al = (row0 // 8) * 8
        offs.append(row0 - al)
        pltpu.sync_copy(scores_hbm.at[pl.ds(al, QT + 7)], bufs[j])

    lane = lax.iota(jnp.int32, 16)
    inf16 = jnp.full((16,), jnp.inf, jnp.float32)
    zero16 = jnp.zeros((16,), jnp.int32)
    sentinel = jnp.where(lane < K, 0, jnp.int32(2**30))

    scope = jax.named_scope

    def per_query(qq, _):
        def per_chunk(c, st):
            out = []
            for j in range(NV):
                bv, bi = st[2 * j], st[2 * j + 1]
                vals = bufs[j][qq + offs[j], pl.ds(c * 16, 16)]
                idxs = c * 16 + lane
                sv, si = plsc.sort_key_val(vals, idxs)
                rv = lax.rev(sv, (0,))
                ri = lax.rev(si, (0,))
                keep = bv <= rv
                mv = jnp.where(keep, bv, rv)
                mi = jnp.where(keep, bi, ri)
                nbv, nbi = plsc.sort_key_val(mv, mi)
                out.extend([nbv, nbi])
            return tuple(out)

        st = (inf16, zero16) * NV
        for c in range(NCH):
            st = per_chunk(c, st)
        for j in range(NV):
            v1 = j + jnp.where(j >= v, 1, 0)
            base = b * M + v1 * N
            keys = st[2 * j + 1] + sentinel
            gidx, _ = plsc.sort_key_val(keys, keys)
            plsc.store_compressed(idx_v.at[pl.ds(qq * (NV * K) + j * K, 16)],
                                  gidx + base, mask=lane < K)
        return 0

    with scope("sc_topk"):
        lax.fori_loop(0, QT, per_query, 0)

    with scope("sc_gather"):
        out0 = q0 * NV * K
        for ch in range(QT // GQ):
            cp = pltpu.async_copy(
                u_hbm.at[idx_v.at[pl.ds(ch * GQ * NV * K, GQ * NV * K)]],
                rows_v, sem)
            cp.wait()
            pltpu.sync_copy(rows_v, ug_hbm.at[pl.ds(out0 + ch * GQ * NV * K,
                                                    GQ * NV * K)])
        tail0 = (QT // GQ) * GQ * NV * K
        cp = pltpu.async_copy(u_hbm.at[idx_v.at[pl.ds(tail0, NV * K)]],
                              rows_tail, sem)
        cp.wait()
        pltpu.sync_copy(rows_tail, ug_hbm.at[pl.ds(out0 + tail0, NV * K)])


def _topk_gather_sc(scores, U):
    scores2 = scores.reshape(B * V * NV * N, NP)
    U2 = U.reshape(B * M, D)
    mesh = plsc.VectorSubcoreMesh(core_axis_name="c", subcore_axis_name="s")
    fn = pl.kernel(
        _sc_topk_gather_body,
        out_type=jax.ShapeDtypeStruct((B * M * NV * K, D), jnp.float32),
        mesh=mesh,
        scratch_types=[
            pltpu.VMEM((QT + 7, NP), jnp.float32),
            pltpu.VMEM((QT + 7, NP), jnp.float32),
            pltpu.VMEM((QT + 7, NP), jnp.float32),
            pltpu.VMEM((QT * NV * K + 8,), jnp.int32),
            pltpu.VMEM((GQ * NV * K, D), jnp.float32),
            pltpu.VMEM((NV * K, D), jnp.float32),
            pltpu.SemaphoreType.DMA,
        ],
        compiler_params=pltpu.CompilerParams(needs_layout_passes=False),
    )
    return fn(scores2, U2)


QC = 49


def _fusion_body(ug_ref, u_ref, p_ref, wf_ref, wo2_ref, be_ref, out_ref):
    ug = ug_ref[0]
    uq = u_ref[0]
    pre = (ug.reshape(QC, NV * K, D) - uq[:, None, :] + be_ref[...][None, None, :])
    edge = 0.5 * pre * (1.0 + lax.erf(pre * (2.0 ** -0.5)))
    logits = jax.lax.dot_general(
        edge.reshape(QC * NV * K, D), wf_ref[...], (((1,), (0,)), ((), ())),
        preferred_element_type=jnp.float32).reshape(QC, NV * K, D)
    mx = jnp.max(logits, axis=1, keepdims=True)
    e = jnp.exp(logits - mx)
    edge_sum = jnp.sum(edge * e, axis=1) / jnp.sum(e, axis=1)
    out_ref[0] = p_ref[0] + jax.lax.dot_general(
        edge_sum, wo2_ref[...], (((1,), (0,)), ((), ())),
        preferred_element_type=jnp.float32)


def _fusion(Ug, U, P, W_fusion, Wo2, b_edge):
    R = B * M
    grid = (R // QC,)
    return pl.pallas_call(
        _fusion_body,
        grid=grid,
        in_specs=[
            pl.BlockSpec((1, QC * NV * K, D), lambda i: (i, 0, 0)),
            pl.BlockSpec((1, QC, D), lambda i: (i, 0, 0)),
            pl.BlockSpec((1, QC, D), lambda i: (i, 0, 0)),
            pl.BlockSpec((D, D), lambda i: (0, 0)),
            pl.BlockSpec((D, D), lambda i: (0, 0)),
            pl.BlockSpec((D,), lambda i: (0,)),
        ],
        out_specs=pl.BlockSpec((1, QC, D), lambda i: (i, 0, 0)),
        out_shape=jax.ShapeDtypeStruct((R // QC, QC, D), jnp.float32),
    )(Ug.reshape(R // QC, QC * NV * K, D),
      U.reshape(R // QC, QC, D),
      P.reshape(R // QC, QC, D),
      W_fusion, Wo2, b_edge)


def kernel(x, W_edge, b_edge, W_fusion, W_offset, b_offset):
    scores, U, P = _scores_proj(x, W_edge, W_offset[:D], b_offset)
    Ug = _topk_gather_sc(scores, U)
    out = _fusion(Ug,
                  U.reshape(B * M, D),
                  P.reshape(B * M, D),
                  W_fusion, W_offset[D:], b_edge)
    return out.reshape(B, V, N, D)


# speedup vs baseline: 2.8572x; 1.0735x over previous
import functools

import jax
import jax.numpy as jnp
from jax import lax
from jax.experimental import pallas as pl
from jax.experimental.pallas import tpu as pltpu
from jax.experimental.pallas import tpu_sc as plsc

B, V, N, D, K = 2, 4, 196, 384, 8
NV = V - 1
M = V * N
NP = 256
QT = 49
NT = B * M // QT
NCH = NP // 16
KQ = QT * K


def _scores_proj_body(xq_ref, xk_ref, we_ref, wo1_ref, bo_ref,
                      out_ref, u16_ref, p_ref):
    xq = xq_ref[0, 0]
    xk = xk_ref[0, 0]
    gram = jax.lax.dot_general(xq, xk, (((1,), (1,)), ((), ())),
                               preferred_element_type=jnp.float32)
    ones_row = jnp.ones((8, D), jnp.float32)
    ss = jax.lax.dot_general(ones_row, xk * xk, (((1,), (1,)), ((), ())),
                             preferred_element_type=jnp.float32)[:1]
    s = ss - 2.0 * gram
    pad = jnp.full((N, NP - N), jnp.inf, jnp.float32)
    out_ref[0, 0, 0] = jnp.concatenate([s, pad], axis=1)

    @pl.when(pl.program_id(2) == 0)
    def _():
        u16_ref[0, 0] = jax.lax.dot_general(
            xq, we_ref[...], (((1,), (0,)), ((), ())),
            preferred_element_type=jnp.float32).astype(jnp.bfloat16)
        p_ref[0, 0] = xq + jax.lax.dot_general(
            xq, wo1_ref[...], (((1,), (0,)), ((), ())),
            preferred_element_type=jnp.float32) + bo_ref[...][None, :]


def _scores_proj(x, W_edge, Wo1, b_offset):
    grid = (B, V, NV)
    return pl.pallas_call(
        _scores_proj_body,
        grid=grid,
        in_specs=[
            pl.BlockSpec((1, 1, N, D), lambda b, v, j: (b, v, 0, 0)),
            pl.BlockSpec((1, 1, N, D),
                         lambda b, v, j: (b, j + (j >= v).astype(j.dtype), 0, 0)),
            pl.BlockSpec((D, D), lambda b, v, j: (0, 0)),
            pl.BlockSpec((D, D), lambda b, v, j: (0, 0)),
            pl.BlockSpec((D,), lambda b, v, j: (0,)),
        ],
        out_specs=[
            pl.BlockSpec((1, 1, 1, N, NP), lambda b, v, j: (b, v, j, 0, 0)),
            pl.BlockSpec((1, 1, N, D), lambda b, v, j: (b, v, 0, 0)),
            pl.BlockSpec((1, 1, N, D), lambda b, v, j: (b, v, 0, 0)),
        ],
        out_shape=[
            jax.ShapeDtypeStruct((B, V, NV, N, NP), jnp.float32),
            jax.ShapeDtypeStruct((B, V, N, D), jnp.bfloat16),
            jax.ShapeDtypeStruct((B, V, N, D), jnp.float32),
        ],
    )(x, x, W_edge, Wo1, b_offset)


def _sc_topk_body(scores_hbm, idx_hbm, b0, b1, b2, idx_v):
    nc = 2
    wid = lax.axis_index("s") * nc + lax.axis_index("c")
    q0 = wid * QT
    b = q0 // M
    v = (q0 % M) // N
    n0 = q0 % N
    bufs = (b0, b1, b2)
    offs = []
    for j in range(NV):
        row0 = ((b * V + v) * NV + j) * N + n0
        al = (row0 // 8) * 8
        offs.append(row0 - al)
        pltpu.sync_copy(scores_hbm.at[pl.ds(al, QT + 7)], bufs[j])

    lane = lax.iota(jnp.int32, 16)
    inf16 = jnp.full((16,), jnp.inf, jnp.float32)
    zero16 = jnp.zeros((16,), jnp.int32)
    sentinel = jnp.where(lane < K, 0, jnp.int32(2**30))

    def per_query(qq, _):
        def per_chunk(c, st):
            out = []
            for j in range(NV):
                bv, bi = st[2 * j], st[2 * j + 1]
                vals = bufs[j][qq + offs[j], pl.ds(c * 16, 16)]
                idxs = c * 16 + lane
                sv, si = plsc.sort_key_val(vals, idxs)
                rv = lax.rev(sv, (0,))
                ri = lax.rev(si, (0,))
                keep = bv <= rv
                mv = jnp.where(keep, bv, rv)
                mi = jnp.where(keep, bi, ri)
                nbv, nbi = plsc.sort_key_val(mv, mi)
                out.extend([nbv, nbi])
            return tuple(out)

        st = (inf16, zero16) * NV
        for c in range(NCH):
            st = per_chunk(c, st)
        for j in range(NV):
            keys = st[2 * j + 1] + sentinel
            gidx, _ = plsc.sort_key_val(keys, keys)
            plsc.store_compressed(idx_v.at[pl.ds(j * KQ + qq * K, 16)],
                                  gidx, mask=lane < K)
        return 0

    lax.fori_loop(0, QT, per_query, 0)
    pltpu.sync_copy(idx_v.at[pl.ds(0, NV * KQ)],
                    idx_hbm.at[pl.ds(wid * NV * KQ, NV * KQ)])


def _topk_sc(scores):
    scores2 = scores.reshape(B * V * NV * N, NP)
    mesh = plsc.VectorSubcoreMesh(core_axis_name="c", subcore_axis_name="s")
    fn = pl.kernel(
        _sc_topk_body,
        out_type=jax.ShapeDtypeStruct((NT * NV * KQ,), jnp.int32),
        mesh=mesh,
        scratch_types=[
            pltpu.VMEM((QT + 7, NP), jnp.float32),
            pltpu.VMEM((QT + 7, NP), jnp.float32),
            pltpu.VMEM((QT + 7, NP), jnp.float32),
            pltpu.VMEM((NV * KQ + 8,), jnp.int32),
        ],
        compiler_params=pltpu.CompilerParams(needs_layout_passes=False),
    )
    return fn(scores2)


def _fusion_body(idx_ref, uq_ref, ut0_ref, ut1_ref, ut2_ref, p_ref,
                 wf_ref, wo2_ref, be_ref, out_ref):
    uts = (ut0_ref, ut1_ref, ut2_ref)
    uq = uq_ref[0].astype(jnp.float32)
    be = be_ref[...]
    iota = lax.broadcasted_iota(jnp.int32, (KQ, N), 1)
    edges, logits_l, mxs = [], [], []
    for j in range(NV):
        idxj = idx_ref[0, j]
        oh = jnp.where(idxj == iota, 1.0, 0.0).astype(jnp.bfloat16)
        ug = jax.lax.dot_general(
            oh, uts[j][0, 0], (((1,), (0,)), ((), ())),
            preferred_element_type=jnp.float32)
        pre = ug.reshape(QT, K, D) - uq[:, None, :] + be[None, None, :]
        edge = 0.5 * pre * (1.0 + lax.erf(pre * (2.0 ** -0.5)))
        lg = jax.lax.dot_general(
            edge.reshape(KQ, D).astype(jnp.bfloat16), wf_ref[...],
            (((1,), (0,)), ((), ())),
            preferred_element_type=jnp.float32).reshape(QT, K, D)
        edges.append(edge)
        logits_l.append(lg)
        mxs.append(jnp.max(lg, axis=1, keepdims=True))
    mx = jnp.maximum(jnp.maximum(mxs[0], mxs[1]), mxs[2])
    acc = jnp.zeros((QT, D), jnp.float32)
    den = jnp.zeros((QT, D), jnp.float32)
    for j in range(NV):
        e = jnp.exp(logits_l[j] - mx)
        acc = acc + jnp.sum(edges[j] * e, axis=1)
        den = den + jnp.sum(e, axis=1)
    edge_sum = acc / den
    out_ref[0] = p_ref[0] + jax.lax.dot_general(
        edge_sum, wo2_ref[...], (((1,), (0,)), ((), ())),
        preferred_element_type=jnp.float32)


def _fusion(idx4, U16q, U16, P, Wf16, Wo2, b_edge):
    grid = (NT,)

    def _ut_spec(j):
        def imap(i):
            v = (i // (N // QT)) % V
            b = i // (V * (N // QT))
            v1 = j + (j >= v).astype(i.dtype)
            return (b, v1, 0, 0)
        return pl.BlockSpec((1, 1, N, D), imap)

    return pl.pallas_call(
        _fusion_body,
        grid=grid,
        in_specs=[
            pl.BlockSpec((1, NV, KQ, 1), lambda i: (i, 0, 0, 0)),
            pl.BlockSpec((1, QT, D), lambda i: (i, 0, 0)),
            _ut_spec(0),
            _ut_spec(1),
            _ut_spec(2),
            pl.BlockSpec((1, QT, D), lambda i: (i, 0, 0)),
            pl.BlockSpec((D, D), lambda i: (0, 0)),
            pl.BlockSpec((D, D), lambda i: (0, 0)),
            pl.BlockSpec((D,), lambda i: (0,)),
        ],
        out_specs=pl.BlockSpec((1, QT, D), lambda i: (i, 0, 0)),
        out_shape=jax.ShapeDtypeStruct((NT, QT, D), jnp.float32),
    )(idx4, U16q, U16, U16, U16, P, Wf16, Wo2, b_edge)


def kernel(x, W_edge, b_edge, W_fusion, W_offset, b_offset):
    scores, U16, P = _scores_proj(x, W_edge, W_offset[:D], b_offset)
    idx = _topk_sc(scores)
    idx4 = idx.reshape(NT, NV, KQ, 1)
    out = _fusion(idx4,
                  U16.reshape(NT, QT, D),
                  U16, P.reshape(NT, QT, D),
                  W_fusion.astype(jnp.bfloat16), W_offset[D:], b_edge)
    return out.reshape(B, V, N, D)


# speedup vs baseline: 3.8333x; 1.3416x over previous
import functools

import numpy as _np

import jax
import jax.numpy as jnp
from jax import lax
from jax.experimental import pallas as pl
from jax.experimental.pallas import tpu as pltpu
from jax.experimental.pallas import tpu_sc as plsc

B, V, N, D, K = 2, 4, 196, 384, 8
NV = V - 1
M = V * N
NP = 256
NPQ = 200
QT = 49
NT = B * M // QT
NCH = NP // 16
KQ = QT * K


def _scores_proj_body(xq_ref, xk_ref, we_ref, wo1_ref, bo_ref,
                      out_ref, u16_ref, p_ref):
    xq = xq_ref[0, 0]
    xk = xk_ref[0, 0]
    gram = jax.lax.dot_general(xq, xk, (((1,), (1,)), ((), ())),
                               preferred_element_type=jnp.float32)
    ones_row = jnp.ones((8, D), jnp.float32)
    ss = jax.lax.dot_general(ones_row, xk * xk, (((1,), (1,)), ((), ())),
                             preferred_element_type=jnp.float32)[:1]
    s = ss - 2.0 * gram
    pad = jnp.full((N, NP - N), jnp.inf, jnp.float32)
    rpad = jnp.full((NPQ - N, NP), jnp.inf, jnp.float32)
    out_ref[0] = jnp.concatenate(
        [jnp.concatenate([s, pad], axis=1), rpad], axis=0)

    @pl.when(pl.program_id(2) == 0)
    def _():
        u16_ref[0, 0] = jax.lax.dot_general(
            xq, we_ref[...], (((1,), (0,)), ((), ())),
            preferred_element_type=jnp.float32).astype(jnp.bfloat16)
        p_ref[0, 0] = xq + jax.lax.dot_general(
            xq, wo1_ref[...], (((1,), (0,)), ((), ())),
            preferred_element_type=jnp.float32) + bo_ref[...][None, :]


def _scores_proj(x, W_edge, Wo1, b_offset):
    grid = (B, V, NV)
    return pl.pallas_call(
        _scores_proj_body,
        grid=grid,
        in_specs=[
            pl.BlockSpec((1, 1, N, D), lambda b, v, j: (b, v, 0, 0)),
            pl.BlockSpec((1, 1, N, D),
                         lambda b, v, j: (b, j + (j >= v).astype(j.dtype), 0, 0)),
            pl.BlockSpec((D, D), lambda b, v, j: (0, 0)),
            pl.BlockSpec((D, D), lambda b, v, j: (0, 0)),
            pl.BlockSpec((D,), lambda b, v, j: (0,)),
        ],
        out_specs=[
            pl.BlockSpec((1, NPQ, NP),
                         lambda b, v, j: ((b * V + v) * NV + j, 0, 0)),
            pl.BlockSpec((1, 1, N, D), lambda b, v, j: (b, v, 0, 0)),
            pl.BlockSpec((1, 1, N, D), lambda b, v, j: (b, v, 0, 0)),
        ],
        out_shape=[
            jax.ShapeDtypeStruct((B * V * NV, NPQ, NP), jnp.float32),
            jax.ShapeDtypeStruct((B, V, N, D), jnp.bfloat16),
            jax.ShapeDtypeStruct((B, V, N, D), jnp.float32),
        ],
    )(x, x, W_edge, Wo1, b_offset)


def _sc_topk_body(scores_hbm, idx_hbm, b0, b1, b2, idx_v):
    nc = 2
    wid = lax.axis_index("s") * nc + lax.axis_index("c")
    q0 = wid * QT
    b = q0 // M
    v = (q0 % M) // N
    n0 = q0 % N
    bufs = (b0, b1, b2)
    offs = []
    for j in range(NV):
        row0 = ((b * V + v) * NV + j) * NPQ + n0
        al = (row0 // 8) * 8
        offs.append(row0 - al)
        pltpu.sync_copy(scores_hbm.at[pl.ds(al, QT + 7)], bufs[j])

    lane = lax.iota(jnp.int32, 16)
    inf16 = jnp.full((16,), jnp.inf, jnp.float32)
    zero16 = jnp.zeros((16,), jnp.int32)
    sentinel = jnp.where(lane < K, 0, jnp.int32(2**30))

    def per_query(qq, _):
        def per_chunk(c, st):
            out = []
            for j in range(NV):
                bv, bi = st[2 * j], st[2 * j + 1]
                vals = bufs[j][qq + offs[j], pl.ds(c * 16, 16)]
                idxs = c * 16 + lane
                sv, si = plsc.sort_key_val(vals, idxs)
                rv = lax.rev(sv, (0,))
                ri = lax.rev(si, (0,))
                keep = bv <= rv
                mv = jnp.where(keep, bv, rv)
                mi = jnp.where(keep, bi, ri)
                nbv, nbi = plsc.sort_key_val(mv, mi)
                out.extend([nbv, nbi])
            return tuple(out)

        st = (inf16, zero16) * NV
        for c in range(NCH):
            st = per_chunk(c, st)
        for j in range(NV):
            keys = st[2 * j + 1] + sentinel
            gidx, _ = plsc.sort_key_val(keys, keys)
            plsc.store_compressed(idx_v.at[pl.ds(j * KQ + qq * K, 16)],
                                  gidx, mask=lane < K)
        return 0

    lax.fori_loop(0, QT, per_query, 0)
    pltpu.sync_copy(idx_v.at[pl.ds(0, NV * KQ)],
                    idx_hbm.at[pl.ds(wid * NV * KQ, NV * KQ)])


def _topk_sc(scores2):
    mesh = plsc.VectorSubcoreMesh(core_axis_name="c", subcore_axis_name="s")
    fn = pl.kernel(
        _sc_topk_body,
        out_type=jax.ShapeDtypeStruct((NT * NV * KQ,), jnp.int32),
        mesh=mesh,
        scratch_types=[
            pltpu.VMEM((QT + 7, NP), jnp.float32),
            pltpu.VMEM((QT + 7, NP), jnp.float32),
            pltpu.VMEM((QT + 7, NP), jnp.float32),
            pltpu.VMEM((NV * KQ + 8,), jnp.int32),
        ],
        compiler_params=pltpu.CompilerParams(needs_layout_passes=False),
    )
    return fn(scores2)


TPB = N // QT
NR = TPB * KQ


def _fusion_body(idx_ref, uq_ref, ut0_ref, ut1_ref, ut2_ref, p_ref,
                 wf_ref, wo2_ref, be_ref, rmat_ref, smat_ref, out_ref):
    uts = (ut0_ref, ut1_ref, ut2_ref)
    iota = lax.broadcasted_iota(jnp.int32, (NR, N), 1)
    rq = jax.lax.dot_general(rmat_ref[...], uq_ref[0, 0],
                             (((1,), (0,)), ((), ())),
                             preferred_element_type=jnp.float32)
    base = be_ref[...][None, :] - rq
    ee_sum = jnp.zeros((NR, D), jnp.float32)
    e_sum = jnp.zeros((NR, D), jnp.float32)
    for j in range(NV):
        col = jnp.concatenate([idx_ref[t, j] for t in range(TPB)], axis=0)
        oh = jnp.where(col == iota, 1.0, 0.0).astype(jnp.bfloat16)
        ug = jax.lax.dot_general(
            oh, uts[j][0, 0], (((1,), (0,)), ((), ())),
            preferred_element_type=jnp.float32)
        pre = ug + base
        edge = 0.5 * pre * (1.0 + lax.erf(pre * (2.0 ** -0.5)))
        lg = jax.lax.dot_general(
            edge.astype(jnp.bfloat16), wf_ref[...], (((1,), (0,)), ((), ())),
            preferred_element_type=jnp.float32)
        e = jnp.exp(lg)
        e_sum = e_sum + e
        ee_sum = ee_sum + edge * e
    both = jnp.concatenate([ee_sum.astype(jnp.bfloat16),
                            e_sum.astype(jnp.bfloat16)], axis=1)
    red = jax.lax.dot_general(smat_ref[...], both, (((1,), (0,)), ((), ())),
                              preferred_element_type=jnp.float32)
    edge_sum = red[:, :D] / red[:, D:]
    out_ref[0, 0] = p_ref[0, 0] + jax.lax.dot_general(
        edge_sum, wo2_ref[...], (((1,), (0,)), ((), ())),
        preferred_element_type=jnp.float32)


def _fusion(idx4, U16, P, Wf16, Wo2, b_edge, Rmat, Smat):
    grid = (B, V)

    def _ut_spec(j):
        def imap(b, v):
            return (b, j + (j >= v).astype(v.dtype), 0, 0)
        return pl.BlockSpec((1, 1, N, D), imap)

    return pl.pallas_call(
        _fusion_body,
        grid=grid,
        in_specs=[
            pl.BlockSpec((TPB, NV, KQ, 1), lambda b, v: (b * V + v, 0, 0, 0)),
            pl.BlockSpec((1, 1, N, D), lambda b, v: (b, v, 0, 0)),
            _ut_spec(0),
            _ut_spec(1),
            _ut_spec(2),
            pl.BlockSpec((1, 1, N, D), lambda b, v: (b, v, 0, 0)),
            pl.BlockSpec((D, D), lambda b, v: (0, 0)),
            pl.BlockSpec((D, D), lambda b, v: (0, 0)),
            pl.BlockSpec((D,), lambda b, v: (0,)),
            pl.BlockSpec((NR, N), lambda b, v: (0, 0)),
            pl.BlockSpec((N, NR), lambda b, v: (0, 0)),
        ],
        out_specs=pl.BlockSpec((1, 1, N, D), lambda b, v: (b, v, 0, 0)),
        out_shape=jax.ShapeDtypeStruct((B, V, N, D), jnp.float32),
    )(idx4, U16, U16, U16, U16, P, Wf16, Wo2, b_edge, Rmat, Smat)


_REP = _np.arange(NR)[:, None] // K == _np.arange(N)[None, :]


def kernel(x, W_edge, b_edge, W_fusion, W_offset, b_offset):
    rmat = jnp.asarray(_REP.astype(_np.float32), dtype=jnp.bfloat16)
    smat = jnp.asarray(_REP.T.astype(_np.float32), dtype=jnp.bfloat16)
    scores3, U16, P = _scores_proj(x, W_edge, W_offset[:D], b_offset)
    idx = _topk_sc(scores3.reshape(B * V * NV * NPQ, NP))
    idx4 = idx.reshape(NT, NV, KQ, 1)
    return _fusion(idx4, U16, P,
                   W_fusion.astype(jnp.bfloat16), W_offset[D:], b_edge,
                   rmat, smat)


# speedup vs baseline: 4.4226x; 1.1537x over previous
import functools

import numpy as _np

import jax
import jax.numpy as jnp
from jax import lax
from jax.experimental import pallas as pl
from jax.experimental.pallas import tpu as pltpu
from jax.experimental.pallas import tpu_sc as plsc

B, V, N, D, K = 2, 4, 196, 384, 8
NV = V - 1
M = V * N
NP = 256
NPQ = 200
QT = 49
NT = B * M // QT
NCH = NP // 16
KQ = QT * K


def _scores_proj_body(xq_ref, xk_ref, we_ref, wo1_ref, bo_ref,
                      out_ref, u16_ref, p_ref):
    xq = xq_ref[0, 0]
    xk = xk_ref[0, 0]
    gram = jax.lax.dot_general(xq, xk, (((1,), (1,)), ((), ())),
                               preferred_element_type=jnp.float32)
    ones_row = jnp.ones((8, D), jnp.float32)
    ss = jax.lax.dot_general(ones_row, xk * xk, (((1,), (1,)), ((), ())),
                             preferred_element_type=jnp.float32)[:1]
    s = ss - 2.0 * gram
    pad = jnp.full((N, NP - N), jnp.inf, jnp.float32)
    rpad = jnp.full((NPQ - N, NP), jnp.inf, jnp.float32)
    out_ref[0] = jnp.concatenate(
        [jnp.concatenate([s, pad], axis=1), rpad], axis=0)

    @pl.when(pl.program_id(2) == 0)
    def _():
        u16_ref[0, 0] = jax.lax.dot_general(
            xq, we_ref[...], (((1,), (0,)), ((), ())),
            preferred_element_type=jnp.float32).astype(jnp.bfloat16)
        p_ref[0, 0] = xq + jax.lax.dot_general(
            xq, wo1_ref[...], (((1,), (0,)), ((), ())),
            preferred_element_type=jnp.float32) + bo_ref[...][None, :]


def _scores_proj(x, W_edge, Wo1, b_offset):
    grid = (B, V, NV)
    return pl.pallas_call(
        _scores_proj_body,
        grid=grid,
        in_specs=[
            pl.BlockSpec((1, 1, N, D), lambda b, v, j: (b, v, 0, 0)),
            pl.BlockSpec((1, 1, N, D),
                         lambda b, v, j: (b, j + (j >= v).astype(j.dtype), 0, 0)),
            pl.BlockSpec((D, D), lambda b, v, j: (0, 0)),
            pl.BlockSpec((D, D), lambda b, v, j: (0, 0)),
            pl.BlockSpec((D,), lambda b, v, j: (0,)),
        ],
        out_specs=[
            pl.BlockSpec((1, NPQ, NP),
                         lambda b, v, j: ((b * V + v) * NV + j, 0, 0)),
            pl.BlockSpec((1, 1, N, D), lambda b, v, j: (b, v, 0, 0)),
            pl.BlockSpec((1, 1, N, D), lambda b, v, j: (b, v, 0, 0)),
        ],
        out_shape=[
            jax.ShapeDtypeStruct((B * V * NV, NPQ, NP), jnp.float32),
            jax.ShapeDtypeStruct((B, V, N, D), jnp.bfloat16),
            jax.ShapeDtypeStruct((B, V, N, D), jnp.float32),
        ],
    )(x, x, W_edge, Wo1, b_offset)


def _sc_topk_body(scores_hbm, idx_hbm, b0, b1, b2, idx_v):
    nc = 2
    wid = lax.axis_index("s") * nc + lax.axis_index("c")
    q0 = wid * QT
    b = q0 // M
    v = (q0 % M) // N
    n0 = q0 % N
    bufs = (b0, b1, b2)
    offs = []
    for j in range(NV):
        row0 = ((b * V + v) * NV + j) * NPQ + n0
        al = (row0 // 8) * 8
        offs.append(row0 - al)
        pltpu.sync_copy(scores_hbm.at[pl.ds(al, QT + 7)], bufs[j])

    lane = lax.iota(jnp.int32, 16)
    inf16 = jnp.full((16,), jnp.inf, jnp.float32)
    zero16 = jnp.zeros((16,), jnp.int32)
    sentinel = jnp.where(lane < K, 0, jnp.int32(2**30))

    def per_query(qq, _):
        def per_chunk(c, st):
            out = []
            for j in range(NV):
                bv, bi = st[2 * j], st[2 * j + 1]
                vals = bufs[j][qq + offs[j], pl.ds(c * 16, 16)]
                idxs = c * 16 + lane
                sv, si = plsc.sort_key_val(vals, idxs)
                rv = lax.rev(sv, (0,))
                ri = lax.rev(si, (0,))
                keep = bv <= rv
                mv = jnp.where(keep, bv, rv)
                mi = jnp.where(keep, bi, ri)
                nbv, nbi = plsc.sort_key_val(mv, mi)
                out.extend([nbv, nbi])
            return tuple(out)

        st = (inf16, zero16) * NV
        for c in range(NCH):
            st = per_chunk(c, st)
        for j in range(NV):
            keys = st[2 * j + 1] + sentinel
            gidx, _ = plsc.sort_key_val(keys, keys)
            plsc.store_compressed(idx_v.at[pl.ds(j * KQ + qq * K, 16)],
                                  gidx, mask=lane < K)
        return 0

    lax.fori_loop(0, QT, per_query, 0)
    pltpu.sync_copy(idx_v.at[pl.ds(0, NV * KQ)],
                    idx_hbm.at[pl.ds(wid * NV * KQ, NV * KQ)])


def _topk_sc(scores2):
    mesh = plsc.VectorSubcoreMesh(core_axis_name="c", subcore_axis_name="s")
    fn = pl.kernel(
        _sc_topk_body,
        out_type=jax.ShapeDtypeStruct((NT * NV * KQ,), jnp.int32),
        mesh=mesh,
        scratch_types=[
            pltpu.VMEM((QT + 7, NP), jnp.float32),
            pltpu.VMEM((QT + 7, NP), jnp.float32),
            pltpu.VMEM((QT + 7, NP), jnp.float32),
            pltpu.VMEM((NV * KQ + 8,), jnp.int32),
        ],
        compiler_params=pltpu.CompilerParams(needs_layout_passes=False),
    )
    return fn(scores2)


TPB = N // QT
NR = TPB * KQ


def _fusion_body(idx_ref, uq_ref, ut0_ref, ut1_ref, ut2_ref, p_ref,
                 wf_ref, wo2_ref, be_ref, rmat_ref, smat_ref, out_ref):
    uts = (ut0_ref, ut1_ref, ut2_ref)
    iota0 = lax.broadcasted_iota(jnp.int32, (N, KQ), 0)
    rq = jax.lax.dot_general(rmat_ref[...], uq_ref[0, 0],
                             (((1,), (0,)), ((), ())),
                             preferred_element_type=jnp.float32)
    base = be_ref[...][None, :] - rq
    ee_sum = jnp.zeros((NR, D), jnp.float32)
    e_sum = jnp.zeros((NR, D), jnp.float32)
    for j in range(NV):
        ug_parts = []
        for t in range(TPB):
            row = idx_ref[t, j].reshape(1, KQ)
            oht = jnp.where(row == iota0, 1.0, 0.0).astype(jnp.bfloat16)
            ug_parts.append(jax.lax.dot_general(
                oht, uts[j][0, 0], (((0,), (0,)), ((), ())),
                preferred_element_type=jnp.float32))
        ug = jnp.concatenate(ug_parts, axis=0)
        pre = ug + base
        edge = 0.5 * pre * (1.0 + lax.erf(pre * (2.0 ** -0.5)))
        lg = jax.lax.dot_general(
            edge.astype(jnp.bfloat16), wf_ref[...], (((1,), (0,)), ((), ())),
            preferred_element_type=jnp.float32)
        e = jnp.exp(lg)
        e_sum = e_sum + e
        ee_sum = ee_sum + edge * e
    both = jnp.concatenate([ee_sum.astype(jnp.bfloat16),
                            e_sum.astype(jnp.bfloat16)], axis=1)
    red = jax.lax.dot_general(smat_ref[...], both, (((1,), (0,)), ((), ())),
                              preferred_element_type=jnp.float32)
    edge_sum = red[:, :D] / red[:, D:]
    out_ref[0, 0] = p_ref[0, 0] + jax.lax.dot_general(
        edge_sum, wo2_ref[...], (((1,), (0,)), ((), ())),
        preferred_element_type=jnp.float32)


def _fusion(idx4, U16, P, Wf16, Wo2, b_edge, Rmat, Smat):
    grid = (B, V)

    def _ut_spec(j):
        def imap(b, v):
            return (b, j + (j >= v).astype(v.dtype), 0, 0)
        return pl.BlockSpec((1, 1, N, D), imap)

    return pl.pallas_call(
        _fusion_body,
        grid=grid,
        in_specs=[
            pl.BlockSpec((TPB, NV, KQ), lambda b, v: (b * V + v, 0, 0)),
            pl.BlockSpec((1, 1, N, D), lambda b, v: (b, v, 0, 0)),
            _ut_spec(0),
            _ut_spec(1),
            _ut_spec(2),
            pl.BlockSpec((1, 1, N, D), lambda b, v: (b, v, 0, 0)),
            pl.BlockSpec((D, D), lambda b, v: (0, 0)),
            pl.BlockSpec((D, D), lambda b, v: (0, 0)),
            pl.BlockSpec((D,), lambda b, v: (0,)),
            pl.BlockSpec((NR, N), lambda b, v: (0, 0)),
            pl.BlockSpec((N, NR), lambda b, v: (0, 0)),
        ],
        out_specs=pl.BlockSpec((1, 1, N, D), lambda b, v: (b, v, 0, 0)),
        out_shape=jax.ShapeDtypeStruct((B, V, N, D), jnp.float32),
    )(idx4, U16, U16, U16, U16, P, Wf16, Wo2, b_edge, Rmat, Smat)


_REP = _np.arange(NR)[:, None] // K == _np.arange(N)[None, :]


def kernel(x, W_edge, b_edge, W_fusion, W_offset, b_offset):
    rmat = jnp.asarray(_REP.astype(_np.float32), dtype=jnp.bfloat16)
    smat = jnp.asarray(_REP.T.astype(_np.float32), dtype=jnp.bfloat16)
    scores3, U16, P = _scores_proj(x, W_edge, W_offset[:D], b_offset)
    idx = _topk_sc(scores3.reshape(B * V * NV * NPQ, NP))
    idx4 = idx.reshape(NT, NV, KQ)
    return _fusion(idx4, U16, P,
                   W_fusion.astype(jnp.bfloat16), W_offset[D:], b_edge,
                   rmat, smat)


# speedup vs baseline: 4.4512x; 1.0065x over previous
import functools

import numpy as _np

import jax
import jax.numpy as jnp
from jax import lax
from jax.experimental import pallas as pl
from jax.experimental.pallas import tpu as pltpu
from jax.experimental.pallas import tpu_sc as plsc

B, V, N, D, K = 2, 4, 196, 384, 8
NV = V - 1
M = V * N
NP = 256
NPQ = 208
WSTG = 40
QT = 49
NT = B * M // QT
NCH = NP // 16
KQ = QT * K
QTB = 28
NTB = M // QTB
KQB = QTB * K


def _scores_proj_body(xq_ref, xk_ref, we_ref, wo1_ref, bo_ref,
                      out_ref, u16_ref, p_ref):
    xq = xq_ref[0, 0]
    xk = xk_ref[0, 0]
    gram = jax.lax.dot_general(xq, xk, (((1,), (1,)), ((), ())),
                               preferred_element_type=jnp.float32)
    ones_row = jnp.ones((8, D), jnp.float32)
    ss = jax.lax.dot_general(ones_row, xk * xk, (((1,), (1,)), ((), ())),
                             preferred_element_type=jnp.float32)[:1]
    s = ss - 2.0 * gram
    pad = jnp.full((N, NP - N), jnp.inf, jnp.float32)
    rpad = jnp.full((NPQ - N, NP), jnp.inf, jnp.float32)
    out_ref[0] = jnp.concatenate(
        [jnp.concatenate([s, pad], axis=1), rpad], axis=0)

    @pl.when(pl.program_id(2) == 0)
    def _():
        u16_ref[0, 0] = jax.lax.dot_general(
            xq, we_ref[...], (((1,), (0,)), ((), ())),
            preferred_element_type=jnp.float32).astype(jnp.bfloat16)
        p_ref[0, 0] = xq + jax.lax.dot_general(
            xq, wo1_ref[...], (((1,), (0,)), ((), ())),
            preferred_element_type=jnp.float32) + bo_ref[...][None, :]


def _scores_proj(x, W_edge, Wo1, b_offset):
    grid = (B, V, NV)
    return pl.pallas_call(
        _scores_proj_body,
        grid=grid,
        in_specs=[
            pl.BlockSpec((1, 1, N, D), lambda b, v, j: (b, v, 0, 0)),
            pl.BlockSpec((1, 1, N, D),
                         lambda b, v, j: (b, j + (j >= v).astype(j.dtype), 0, 0)),
            pl.BlockSpec((D, D), lambda b, v, j: (0, 0)),
            pl.BlockSpec((D, D), lambda b, v, j: (0, 0)),
            pl.BlockSpec((D,), lambda b, v, j: (0,)),
        ],
        out_specs=[
            pl.BlockSpec((1, NPQ, NP),
                         lambda b, v, j: ((b * V + v) * NV + j, 0, 0)),
            pl.BlockSpec((1, 1, N, D), lambda b, v, j: (b, v, 0, 0)),
            pl.BlockSpec((1, 1, N, D), lambda b, v, j: (b, v, 0, 0)),
        ],
        out_shape=[
            jax.ShapeDtypeStruct((B * V * NV, NPQ, NP), jnp.float32),
            jax.ShapeDtypeStruct((B, V, N, D), jnp.bfloat16),
            jax.ShapeDtypeStruct((B, V, N, D), jnp.float32),
        ],
    )(x, x, W_edge, Wo1, b_offset)


def _sc_topk_body(scores_hbm, idx_hbm, b0, b1, b2, idx_v, *, bb):
    nc = 2
    wid = lax.axis_index("s") * nc + lax.axis_index("c")

    @pl.when(wid < NTB)
    def _():
        q0 = wid * QTB
        v = q0 // N
        n0 = q0 % N
        bufs = (b0, b1, b2)
        offs = []
        for j in range(NV):
            row0 = ((bb * V + v) * NV + j) * NPQ + n0
            al = (row0 // 8) * 8
            offs.append(row0 - al)
            pltpu.sync_copy(scores_hbm.at[pl.ds(al, WSTG)], bufs[j])

        lane = lax.iota(jnp.int32, 16)
        inf16 = jnp.full((16,), jnp.inf, jnp.float32)
        zero16 = jnp.zeros((16,), jnp.int32)
        sentinel = jnp.where(lane < K, 0, jnp.int32(2**30))

        def per_query(qq, _):
            def per_chunk(c, st):
                out = []
                for j in range(NV):
                    bv, bi = st[2 * j], st[2 * j + 1]
                    vals = bufs[j][qq + offs[j], pl.ds(c * 16, 16)]
                    idxs = c * 16 + lane
                    sv, si = plsc.sort_key_val(vals, idxs)
                    rv = lax.rev(sv, (0,))
                    ri = lax.rev(si, (0,))
                    keep = bv <= rv
                    mv = jnp.where(keep, bv, rv)
                    mi = jnp.where(keep, bi, ri)
                    nbv, nbi = plsc.sort_key_val(mv, mi)
                    out.extend([nbv, nbi])
                return tuple(out)

            st = (inf16, zero16) * NV
            for c in range(NCH):
                st = per_chunk(c, st)
            for j in range(NV):
                keys = st[2 * j + 1] + sentinel
                gidx, _ = plsc.sort_key_val(keys, keys)
                plsc.store_compressed(idx_v.at[pl.ds(j * KQB + qq * K, 16)],
                                      gidx, mask=lane < K)
            return 0

        lax.fori_loop(0, QTB, per_query, 0)
        pltpu.sync_copy(idx_v.at[pl.ds(0, NV * KQB)],
                        idx_hbm.at[pl.ds(wid * NV * KQB, NV * KQB)])


def _topk_sc(scores2, b):
    mesh = plsc.VectorSubcoreMesh(core_axis_name="c", subcore_axis_name="s")
    fn = pl.kernel(
        functools.partial(_sc_topk_body, bb=b),
        out_type=jax.ShapeDtypeStruct((NTB * NV * KQB,), jnp.int32),
        mesh=mesh,
        scratch_types=[
            pltpu.VMEM((WSTG, NP), jnp.float32),
            pltpu.VMEM((WSTG, NP), jnp.float32),
            pltpu.VMEM((WSTG, NP), jnp.float32),
            pltpu.VMEM((NV * KQB + 8,), jnp.int32),
        ],
        compiler_params=pltpu.CompilerParams(needs_layout_passes=False),
    )
    return fn(scores2)


TPB = N // QTB
NR = N * K


def _fusion_body(idx_ref, uq_ref, ut0_ref, ut1_ref, ut2_ref, p_ref,
                 wf_ref, wo2_ref, be_ref, rmat_ref, smat_ref, out_ref):
    uts = (ut0_ref, ut1_ref, ut2_ref)
    iota0 = lax.broadcasted_iota(jnp.int32, (N, KQB), 0)
    rq = jax.lax.dot_general(rmat_ref[...], uq_ref[0, 0],
                             (((1,), (0,)), ((), ())),
                             preferred_element_type=jnp.float32)
    base = be_ref[...][None, :] - rq
    ee_sum = jnp.zeros((NR, D), jnp.float32)
    e_sum = jnp.zeros((NR, D), jnp.float32)
    for j in range(NV):
        ug_parts = []
        for t in range(TPB):
            row = idx_ref[t, j].reshape(1, KQB)
            oht = jnp.where(row == iota0, 1.0, 0.0).astype(jnp.bfloat16)
            ug_parts.append(jax.lax.dot_general(
                oht, uts[j][0, 0], (((0,), (0,)), ((), ())),
                preferred_element_type=jnp.float32))
        ug = jnp.concatenate(ug_parts, axis=0)
        pre = ug + base
        edge = 0.5 * pre * (1.0 + lax.erf(pre * (2.0 ** -0.5)))
        lg = jax.lax.dot_general(
            edge.astype(jnp.bfloat16), wf_ref[...], (((1,), (0,)), ((), ())),
            preferred_element_type=jnp.float32)
        e = jnp.exp(lg)
        e_sum = e_sum + e
        ee_sum = ee_sum + edge * e
    both = jnp.concatenate([ee_sum.astype(jnp.bfloat16),
                            e_sum.astype(jnp.bfloat16)], axis=1)
    red = jax.lax.dot_general(smat_ref[...], both, (((1,), (0,)), ((), ())),
                              preferred_element_type=jnp.float32)
    edge_sum = red[:, :D] / red[:, D:]
    out_ref[0] = p_ref[0, 0] + jax.lax.dot_general(
        edge_sum, wo2_ref[...], (((1,), (0,)), ((), ())),
        preferred_element_type=jnp.float32)


def _fusion(idx3, U16, P, Wf16, Wo2, b_edge, Rmat, Smat, b):
    grid = (V,)

    def _ut_spec(j):
        def imap(v):
            return (b, j + (j >= v).astype(v.dtype), 0, 0)
        return pl.BlockSpec((1, 1, N, D), imap)

    return pl.pallas_call(
        _fusion_body,
        grid=grid,
        in_specs=[
            pl.BlockSpec((TPB, NV, KQB), lambda v: (v, 0, 0)),
            pl.BlockSpec((1, 1, N, D), lambda v: (b, v, 0, 0)),
            _ut_spec(0),
            _ut_spec(1),
            _ut_spec(2),
            pl.BlockSpec((1, 1, N, D), lambda v: (b, v, 0, 0)),
            pl.BlockSpec((D, D), lambda v: (0, 0)),
            pl.BlockSpec((D, D), lambda v: (0, 0)),
            pl.BlockSpec((D,), lambda v: (0,)),
            pl.BlockSpec((NR, N), lambda v: (0, 0)),
            pl.BlockSpec((N, NR), lambda v: (0, 0)),
        ],
        out_specs=pl.BlockSpec((1, N, D), lambda v: (v, 0, 0)),
        out_shape=jax.ShapeDtypeStruct((V, N, D), jnp.float32),
    )(idx3, U16, U16, U16, U16, P, Wf16, Wo2, b_edge, Rmat, Smat)


_REP = _np.arange(NR)[:, None] // K == _np.arange(N)[None, :]


def kernel(x, W_edge, b_edge, W_fusion, W_offset, b_offset):
    rmat = jnp.asarray(_REP.astype(_np.float32), dtype=jnp.bfloat16)
    smat = jnp.asarray(_REP.T.astype(_np.float32), dtype=jnp.bfloat16)
    wf16 = W_fusion.astype(jnp.bfloat16)
    scores3, U16, P = _scores_proj(x, W_edge, W_offset[:D], b_offset)
    scores2 = scores3.reshape(B * V * NV * NPQ, NP)
    idx = [_topk_sc(scores2, b).reshape(NTB, NV, KQB) for b in range(B)]
    outs = [_fusion(idx[b], U16, P, wf16, W_offset[D:], b_edge,
                    rmat, smat, b) for b in range(B)]
    return jnp.stack(outs, axis=0)
